# baseline (device time: 47002 ns/iter reference)
import jax
import jax.numpy as jnp
from jax import lax
from jax.experimental import pallas as pl
from jax.experimental.pallas import tpu as pltpu

N_DEV = 4
EPS = 1e-6


def kernel(partial, gamma):
    _, m_total, d = partial.shape
    m_per = m_total // N_DEV
    gamma2 = gamma.reshape(1, d)

    def body(p_ref, g_ref, out_ref, send_buf, recv_buf, send_sems, recv_sems):
        my_x = lax.axis_index("x")
        my_y = lax.axis_index("y")
        my_z = lax.axis_index("z")
        left_z = (my_z - 1) % N_DEV
        right_z = (my_z + 1) % N_DEV

        barrier_sem = pltpu.get_barrier_semaphore()
        for nbr_z in (left_z, right_z):
            pl.semaphore_signal(
                barrier_sem,
                inc=1,
                device_id=(my_x, my_y, nbr_z),
                device_id_type=pl.DeviceIdType.MESH,
            )
        pl.semaphore_wait(barrier_sem, 2)

        c0 = (my_z - 1) % N_DEV
        send_buf[:, :] = p_ref[0, pl.ds(c0 * m_per, m_per), :]
        for s in range(N_DEV - 1):
            rdma = pltpu.make_async_remote_copy(
                src_ref=send_buf,
                dst_ref=recv_buf.at[s],
                send_sem=send_sems.at[s],
                recv_sem=recv_sems.at[s],
                device_id=(my_x, my_y, right_z),
                device_id_type=pl.DeviceIdType.MESH,
            )
            rdma.start()
            rdma.wait()
            c = (my_z - s - 2) % N_DEV
            acc = recv_buf[s] + p_ref[0, pl.ds(c * m_per, m_per), :]
            if s < N_DEV - 2:
                send_buf[:, :] = acc
            else:
                rms = jnp.sqrt(
                    jnp.mean(acc * acc, axis=-1, keepdims=True) + EPS
                )
                out_ref[:, :] = acc / rms * g_ref[:, :]

    return pl.pallas_call(
        body,
        out_shape=jax.ShapeDtypeStruct((m_per, d), jnp.float32),
        in_specs=[
            pl.BlockSpec(memory_space=pltpu.VMEM),
            pl.BlockSpec(memory_space=pltpu.VMEM),
        ],
        out_specs=pl.BlockSpec(memory_space=pltpu.VMEM),
        scratch_shapes=[
            pltpu.VMEM((m_per, d), jnp.float32),
            pltpu.VMEM((N_DEV - 1, m_per, d), jnp.float32),
            pltpu.SemaphoreType.DMA((N_DEV - 1,)),
            pltpu.SemaphoreType.DMA((N_DEV - 1,)),
        ],
        compiler_params=pltpu.CompilerParams(collective_id=0),
    )(partial, gamma2)


# device time: 46078 ns/iter; 1.0201x vs baseline; 1.0201x over previous
import jax
import jax.numpy as jnp
from jax import lax
from jax.experimental import pallas as pl
from jax.experimental.pallas import tpu as pltpu

N_DEV = 4
EPS = 1e-6


def kernel(partial, gamma):
    _, m_total, d = partial.shape
    m_per = m_total // N_DEV
    half = d // 2
    gamma2 = gamma.reshape(1, d)

    def body(
        p_ref,
        g_ref,
        out_ref,
        send_cw,
        send_ccw,
        recv_cw,
        recv_ccw,
        send_sems_cw,
        recv_sems_cw,
        send_sems_ccw,
        recv_sems_ccw,
    ):
        my_x = lax.axis_index("x")
        my_y = lax.axis_index("y")
        my_z = lax.axis_index("z")
        left_z = (my_z - 1) % N_DEV
        right_z = (my_z + 1) % N_DEV

        barrier_sem = pltpu.get_barrier_semaphore()
        for nbr_z in (left_z, right_z):
            pl.semaphore_signal(
                barrier_sem,
                inc=1,
                device_id=(my_x, my_y, nbr_z),
                device_id_type=pl.DeviceIdType.MESH,
            )
        pl.semaphore_wait(barrier_sem, 2)

        c_cw0 = (my_z - 1) % N_DEV
        c_ccw0 = (my_z + 1) % N_DEV
        send_cw[:, :] = p_ref[0, pl.ds(c_cw0 * m_per, m_per), 0:half]
        send_ccw[:, :] = p_ref[0, pl.ds(c_ccw0 * m_per, m_per), half:d]
        for s in range(N_DEV - 1):
            rdma_cw = pltpu.make_async_remote_copy(
                src_ref=send_cw,
                dst_ref=recv_cw.at[s],
                send_sem=send_sems_cw.at[s],
                recv_sem=recv_sems_cw.at[s],
                device_id=(my_x, my_y, right_z),
                device_id_type=pl.DeviceIdType.MESH,
            )
            rdma_ccw = pltpu.make_async_remote_copy(
                src_ref=send_ccw,
                dst_ref=recv_ccw.at[s],
                send_sem=send_sems_ccw.at[s],
                recv_sem=recv_sems_ccw.at[s],
                device_id=(my_x, my_y, left_z),
                device_id_type=pl.DeviceIdType.MESH,
            )
            rdma_cw.start()
            rdma_ccw.start()
            rdma_cw.wait()
            rdma_ccw.wait()
            c_cw = (my_z - s - 2) % N_DEV
            c_ccw = (my_z + s + 2) % N_DEV
            acc_cw = recv_cw[s] + p_ref[0, pl.ds(c_cw * m_per, m_per), 0:half]
            acc_ccw = recv_ccw[s] + p_ref[0, pl.ds(c_ccw * m_per, m_per), half:d]
            if s < N_DEV - 2:
                send_cw[:, :] = acc_cw
                send_ccw[:, :] = acc_ccw
            else:
                sumsq = jnp.sum(acc_cw * acc_cw, axis=1, keepdims=True) + jnp.sum(
                    acc_ccw * acc_ccw, axis=1, keepdims=True
                )
                rms = jnp.sqrt(sumsq / d + EPS)
                out_ref[:, 0:half] = acc_cw / rms * g_ref[:, 0:half]
                out_ref[:, half:d] = acc_ccw / rms * g_ref[:, half:d]

    return pl.pallas_call(
        body,
        out_shape=jax.ShapeDtypeStruct((m_per, d), jnp.float32),
        in_specs=[
            pl.BlockSpec(memory_space=pltpu.VMEM),
            pl.BlockSpec(memory_space=pltpu.VMEM),
        ],
        out_specs=pl.BlockSpec(memory_space=pltpu.VMEM),
        scratch_shapes=[
            pltpu.VMEM((m_per, half), jnp.float32),
            pltpu.VMEM((m_per, half), jnp.float32),
            pltpu.VMEM((N_DEV - 1, m_per, half), jnp.float32),
            pltpu.VMEM((N_DEV - 1, m_per, half), jnp.float32),
            pltpu.SemaphoreType.DMA((N_DEV - 1,)),
            pltpu.SemaphoreType.DMA((N_DEV - 1,)),
            pltpu.SemaphoreType.DMA((N_DEV - 1,)),
            pltpu.SemaphoreType.DMA((N_DEV - 1,)),
        ],
        compiler_params=pltpu.CompilerParams(collective_id=0),
    )(partial, gamma2)


# device time: 34707 ns/iter; 1.3543x vs baseline; 1.3276x over previous
import jax
import jax.numpy as jnp
from jax import lax
from jax.experimental import pallas as pl
from jax.experimental.pallas import tpu as pltpu

N_DEV = 4
EPS = 1e-6


def kernel(partial, gamma):
    _, m_total, d = partial.shape
    m_per = m_total // N_DEV
    xhalf = d // 2
    quarter = xhalf // 2
    gamma2 = gamma.reshape(1, d)

    my_x_outer = lax.axis_index("x")
    p_half = lax.dynamic_slice_in_dim(partial, my_x_outer * xhalf, xhalf, axis=2)

    def body(
        p_ref,
        g_ref,
        out_ref,
        send_cw,
        send_ccw,
        recv_cw,
        recv_ccw,
        xbuf,
        send_sems_cw,
        recv_sems_cw,
        send_sems_ccw,
        recv_sems_ccw,
        xsend_sem,
        xrecv_sem,
    ):
        my_x = lax.axis_index("x")
        my_y = lax.axis_index("y")
        my_z = lax.axis_index("z")
        left_z = (my_z - 1) % N_DEV
        right_z = (my_z + 1) % N_DEV
        other_x = 1 - my_x

        barrier_sem = pltpu.get_barrier_semaphore()
        for dev in (
            (my_x, my_y, left_z),
            (my_x, my_y, right_z),
            (other_x, my_y, my_z),
        ):
            pl.semaphore_signal(
                barrier_sem,
                inc=1,
                device_id=dev,
                device_id_type=pl.DeviceIdType.MESH,
            )
        pl.semaphore_wait(barrier_sem, 3)

        c_cw0 = (my_z - 1) % N_DEV
        c_ccw0 = (my_z + 1) % N_DEV
        send_cw[:, :] = p_ref[0, pl.ds(c_cw0 * m_per, m_per), 0:quarter]
        send_ccw[:, :] = p_ref[0, pl.ds(c_ccw0 * m_per, m_per), quarter:xhalf]
        for s in range(N_DEV - 1):
            rdma_cw = pltpu.make_async_remote_copy(
                src_ref=send_cw,
                dst_ref=recv_cw.at[s],
                send_sem=send_sems_cw.at[s],
                recv_sem=recv_sems_cw.at[s],
                device_id=(my_x, my_y, right_z),
                device_id_type=pl.DeviceIdType.MESH,
            )
            rdma_ccw = pltpu.make_async_remote_copy(
                src_ref=send_ccw,
                dst_ref=recv_ccw.at[s],
                send_sem=send_sems_ccw.at[s],
                recv_sem=recv_sems_ccw.at[s],
                device_id=(my_x, my_y, left_z),
                device_id_type=pl.DeviceIdType.MESH,
            )
            rdma_cw.start()
            rdma_ccw.start()
            rdma_cw.wait()
            rdma_ccw.wait()
            c_cw = (my_z - s - 2) % N_DEV
            c_ccw = (my_z + s + 2) % N_DEV
            acc_cw = recv_cw[s] + p_ref[0, pl.ds(c_cw * m_per, m_per), 0:quarter]
            acc_ccw = (
                recv_ccw[s] + p_ref[0, pl.ds(c_ccw * m_per, m_per), quarter:xhalf]
            )
            if s < N_DEV - 2:
                send_cw[:, :] = acc_cw
                send_ccw[:, :] = acc_ccw
            else:
                xbuf[0, :, 0:quarter] = acc_cw
                xbuf[0, :, quarter:xhalf] = acc_ccw

        xch = pltpu.make_async_remote_copy(
            src_ref=xbuf.at[0],
            dst_ref=xbuf.at[1],
            send_sem=xsend_sem,
            recv_sem=xrecv_sem,
            device_id=(other_x, my_y, my_z),
            device_id_type=pl.DeviceIdType.MESH,
        )
        xch.start()
        xch.wait()

        mine = xbuf[0]
        theirs = xbuf[1]
        sumsq = jnp.sum(mine * mine, axis=1, keepdims=True) + jnp.sum(
            theirs * theirs, axis=1, keepdims=True
        )
        rms = jnp.sqrt(sumsq / d + EPS)

        @pl.when(my_x == 0)
        def _():
            out_ref[:, 0:xhalf] = mine / rms * g_ref[:, 0:xhalf]
            out_ref[:, xhalf:d] = theirs / rms * g_ref[:, xhalf:d]

        @pl.when(my_x == 1)
        def _():
            out_ref[:, 0:xhalf] = theirs / rms * g_ref[:, 0:xhalf]
            out_ref[:, xhalf:d] = mine / rms * g_ref[:, xhalf:d]

    return pl.pallas_call(
        body,
        out_shape=jax.ShapeDtypeStruct((m_per, d), jnp.float32),
        in_specs=[
            pl.BlockSpec(memory_space=pltpu.VMEM),
            pl.BlockSpec(memory_space=pltpu.VMEM),
        ],
        out_specs=pl.BlockSpec(memory_space=pltpu.VMEM),
        scratch_shapes=[
            pltpu.VMEM((m_per, quarter), jnp.float32),
            pltpu.VMEM((m_per, quarter), jnp.float32),
            pltpu.VMEM((N_DEV - 1, m_per, quarter), jnp.float32),
            pltpu.VMEM((N_DEV - 1, m_per, quarter), jnp.float32),
            pltpu.VMEM((2, m_per, xhalf), jnp.float32),
            pltpu.SemaphoreType.DMA((N_DEV - 1,)),
            pltpu.SemaphoreType.DMA((N_DEV - 1,)),
            pltpu.SemaphoreType.DMA((N_DEV - 1,)),
            pltpu.SemaphoreType.DMA((N_DEV - 1,)),
            pltpu.SemaphoreType.DMA(()),
            pltpu.SemaphoreType.DMA(()),
        ],
        compiler_params=pltpu.CompilerParams(collective_id=0),
    )(p_half, gamma2)


# device time: 26673 ns/iter; 1.7622x vs baseline; 1.3012x over previous
import jax
import jax.numpy as jnp
from jax import lax
from jax.experimental import pallas as pl
from jax.experimental.pallas import tpu as pltpu

NZ = 4
NY = 4
EPS = 1e-6


def kernel(partial, gamma):
    _, m_total, d = partial.shape
    m_per = m_total // NZ
    m_strip = m_per // (2 * NY)
    gamma2 = gamma.reshape(1, d)
    p2 = partial.reshape(m_total, d)

    def body(
        p_ref,
        g_ref,
        out_ref,
        zrecv,
        zsend_sems,
        zrecv_sems,
        ysend_sems,
        yrecv_sems,
        xsend_sems,
        xrecv_sems,
    ):
        my_x = lax.axis_index("x")
        my_y = lax.axis_index("y")
        my_z = lax.axis_index("z")
        other_x = 1 - my_x
        s_idx = my_x * NY + my_y
        my_rows = pl.ds(s_idx * m_strip, m_strip)

        barrier_sem = pltpu.get_barrier_semaphore()
        for r in range(1, NZ):
            pl.semaphore_signal(
                barrier_sem,
                inc=1,
                device_id=(my_x, my_y, (my_z + r) % NZ),
                device_id_type=pl.DeviceIdType.MESH,
            )
        for r in range(1, NY):
            pl.semaphore_signal(
                barrier_sem,
                inc=1,
                device_id=(my_x, (my_y + r) % NY, my_z),
                device_id_type=pl.DeviceIdType.MESH,
            )
        pl.semaphore_signal(
            barrier_sem,
            inc=1,
            device_id=(other_x, my_y, my_z),
            device_id_type=pl.DeviceIdType.MESH,
        )
        pl.semaphore_wait(barrier_sem, NZ - 1 + NY - 1 + 1)

        zflows = []
        for r in range(1, NZ):
            c = (my_z + r) % NZ
            rdma = pltpu.make_async_remote_copy(
                src_ref=p_ref.at[pl.ds(c * m_per + s_idx * m_strip, m_strip), :],
                dst_ref=zrecv.at[r - 1],
                send_sem=zsend_sems.at[r - 1],
                recv_sem=zrecv_sems.at[r - 1],
                device_id=(my_x, my_y, c),
                device_id_type=pl.DeviceIdType.MESH,
            )
            rdma.start()
            zflows.append(rdma)

        acc = p_ref[pl.ds(my_z * m_per + s_idx * m_strip, m_strip), :]
        for r in range(1, NZ):
            zflows[r - 1].wait_recv()
            acc = acc + zrecv[r - 1]

        rms = jnp.sqrt(jnp.mean(acc * acc, axis=1, keepdims=True) + EPS)
        out_ref[my_rows, :] = acc / rms * g_ref[:, :]

        yflows = []
        for r in range(1, NY):
            rdma = pltpu.make_async_remote_copy(
                src_ref=out_ref.at[my_rows, :],
                dst_ref=out_ref.at[my_rows, :],
                send_sem=ysend_sems.at[r - 1],
                recv_sem=yrecv_sems.at[r - 1],
                device_id=(my_x, (my_y + r) % NY, my_z),
                device_id_type=pl.DeviceIdType.MESH,
            )
            rdma.start()
            yflows.append(rdma)
        xflows = []
        x0 = pltpu.make_async_remote_copy(
            src_ref=out_ref.at[my_rows, :],
            dst_ref=out_ref.at[my_rows, :],
            send_sem=xsend_sems.at[0],
            recv_sem=xrecv_sems.at[0],
            device_id=(other_x, my_y, my_z),
            device_id_type=pl.DeviceIdType.MESH,
        )
        x0.start()
        xflows.append(x0)

        for r in range(1, NY):
            yflows[r - 1].wait_recv()
            src_y = (my_y - r) % NY
            rows = pl.ds((my_x * NY + src_y) * m_strip, m_strip)
            fwd = pltpu.make_async_remote_copy(
                src_ref=out_ref.at[rows, :],
                dst_ref=out_ref.at[rows, :],
                send_sem=xsend_sems.at[r],
                recv_sem=xrecv_sems.at[r],
                device_id=(other_x, my_y, my_z),
                device_id_type=pl.DeviceIdType.MESH,
            )
            fwd.start()
            xflows.append(fwd)

        for k in range(4):
            xflows[k].wait_recv()

        for r in range(1, NZ):
            zflows[r - 1].wait_send()
        for r in range(1, NY):
            yflows[r - 1].wait_send()
        for k in range(4):
            xflows[k].wait_send()

    return pl.pallas_call(
        body,
        out_shape=jax.ShapeDtypeStruct((m_per, d), jnp.float32),
        in_specs=[
            pl.BlockSpec(memory_space=pltpu.VMEM),
            pl.BlockSpec(memory_space=pltpu.VMEM),
        ],
        out_specs=pl.BlockSpec(memory_space=pltpu.VMEM),
        scratch_shapes=[
            pltpu.VMEM((NZ - 1, m_strip, d), jnp.float32),
            pltpu.SemaphoreType.DMA((NZ - 1,)),
            pltpu.SemaphoreType.DMA((NZ - 1,)),
            pltpu.SemaphoreType.DMA((NY - 1,)),
            pltpu.SemaphoreType.DMA((NY - 1,)),
            pltpu.SemaphoreType.DMA((4,)),
            pltpu.SemaphoreType.DMA((4,)),
        ],
        compiler_params=pltpu.CompilerParams(collective_id=0),
    )(p2, gamma2)


# device time: 24437 ns/iter; 1.9234x vs baseline; 1.0915x over previous
import jax
import jax.numpy as jnp
from jax import lax
from jax.experimental import pallas as pl
from jax.experimental.pallas import tpu as pltpu

NZ = 4
NY = 4
NW = 2
EPS = 1e-6


def kernel(partial, gamma):
    _, m_total, d = partial.shape
    m_per = m_total // NZ
    m_strip = m_per // (2 * NY)
    m_wave = m_strip // NW
    gamma2 = gamma.reshape(1, d)
    p2 = partial.reshape(m_total, d)

    def body(
        p_ref,
        g_ref,
        out_ref,
        zrecv,
        zsend_sems,
        zrecv_sems,
        ysend_sems,
        yrecv_sems,
        xsend_sems,
        xrecv_sems,
    ):
        my_x = lax.axis_index("x")
        my_y = lax.axis_index("y")
        my_z = lax.axis_index("z")
        other_x = 1 - my_x
        s_idx = my_x * NY + my_y

        barrier_sem = pltpu.get_barrier_semaphore()
        for r in range(1, NZ):
            pl.semaphore_signal(
                barrier_sem,
                inc=1,
                device_id=(my_x, my_y, (my_z + r) % NZ),
                device_id_type=pl.DeviceIdType.MESH,
            )
        for r in range(1, NY):
            pl.semaphore_signal(
                barrier_sem,
                inc=1,
                device_id=(my_x, (my_y + r) % NY, my_z),
                device_id_type=pl.DeviceIdType.MESH,
            )
        pl.semaphore_signal(
            barrier_sem,
            inc=1,
            device_id=(other_x, my_y, my_z),
            device_id_type=pl.DeviceIdType.MESH,
        )
        pl.semaphore_wait(barrier_sem, NZ - 1 + NY - 1 + 1)

        zflows = [[], []]
        for w in range(NW):
            for r in range(1, NZ):
                c = (my_z + r) % NZ
                src_rows = pl.ds(
                    c * m_per + s_idx * m_strip + w * m_wave, m_wave
                )
                rdma = pltpu.make_async_remote_copy(
                    src_ref=p_ref.at[src_rows, :],
                    dst_ref=zrecv.at[w, r - 1],
                    send_sem=zsend_sems.at[w, r - 1],
                    recv_sem=zrecv_sems.at[w, r - 1],
                    device_id=(my_x, my_y, c),
                    device_id_type=pl.DeviceIdType.MESH,
                )
                rdma.start()
                zflows[w].append(rdma)

        yflows = [[], []]
        xflows = [[], []]
        for w in range(NW):
            wave_rows = pl.ds(s_idx * m_strip + w * m_wave, m_wave)
            acc = p_ref[
                pl.ds(my_z * m_per + s_idx * m_strip + w * m_wave, m_wave), :
            ]
            for r in range(1, NZ):
                zflows[w][r - 1].wait_recv()
                acc = acc + zrecv[w, r - 1]
            rms = jnp.sqrt(jnp.mean(acc * acc, axis=1, keepdims=True) + EPS)
            out_ref[wave_rows, :] = acc / rms * g_ref[:, :]

            for r in range(1, NY):
                rdma = pltpu.make_async_remote_copy(
                    src_ref=out_ref.at[wave_rows, :],
                    dst_ref=out_ref.at[wave_rows, :],
                    send_sem=ysend_sems.at[w, r - 1],
                    recv_sem=yrecv_sems.at[w, r - 1],
                    device_id=(my_x, (my_y + r) % NY, my_z),
                    device_id_type=pl.DeviceIdType.MESH,
                )
                rdma.start()
                yflows[w].append(rdma)
            x0 = pltpu.make_async_remote_copy(
                src_ref=out_ref.at[wave_rows, :],
                dst_ref=out_ref.at[wave_rows, :],
                send_sem=xsend_sems.at[w, 0],
                recv_sem=xrecv_sems.at[w, 0],
                device_id=(other_x, my_y, my_z),
                device_id_type=pl.DeviceIdType.MESH,
            )
            x0.start()
            xflows[w].append(x0)

        for w in range(NW):
            for r in range(1, NY):
                yflows[w][r - 1].wait_recv()
                src_y = (my_y - r) % NY
                rows = pl.ds(
                    (my_x * NY + src_y) * m_strip + w * m_wave, m_wave
                )
                fwd = pltpu.make_async_remote_copy(
                    src_ref=out_ref.at[rows, :],
                    dst_ref=out_ref.at[rows, :],
                    send_sem=xsend_sems.at[w, r],
                    recv_sem=xrecv_sems.at[w, r],
                    device_id=(other_x, my_y, my_z),
                    device_id_type=pl.DeviceIdType.MESH,
                )
                fwd.start()
                xflows[w].append(fwd)

        for w in range(NW):
            for k in range(4):
                xflows[w][k].wait_recv()
        for w in range(NW):
            for r in range(1, NZ):
                zflows[w][r - 1].wait_send()
            for r in range(1, NY):
                yflows[w][r - 1].wait_send()
            for k in range(4):
                xflows[w][k].wait_send()

    return pl.pallas_call(
        body,
        out_shape=jax.ShapeDtypeStruct((m_per, d), jnp.float32),
        in_specs=[
            pl.BlockSpec(memory_space=pltpu.VMEM),
            pl.BlockSpec(memory_space=pltpu.VMEM),
        ],
        out_specs=pl.BlockSpec(memory_space=pltpu.VMEM),
        scratch_shapes=[
            pltpu.VMEM((NW, NZ - 1, m_wave, d), jnp.float32),
            pltpu.SemaphoreType.DMA((NW, NZ - 1)),
            pltpu.SemaphoreType.DMA((NW, NZ - 1)),
            pltpu.SemaphoreType.DMA((NW, NY - 1)),
            pltpu.SemaphoreType.DMA((NW, NY - 1)),
            pltpu.SemaphoreType.DMA((NW, 4)),
            pltpu.SemaphoreType.DMA((NW, 4)),
        ],
        compiler_params=pltpu.CompilerParams(collective_id=0),
    )(p2, gamma2)


# device time: 24061 ns/iter; 1.9535x vs baseline; 1.0156x over previous
import jax
import jax.numpy as jnp
from jax import lax
from jax.experimental import pallas as pl
from jax.experimental.pallas import tpu as pltpu

NZ = 4
NY = 4
NW = 4
EPS = 1e-6


def kernel(partial, gamma):
    _, m_total, d = partial.shape
    m_per = m_total // NZ
    m_strip = m_per // (2 * NY)
    m_wave = m_strip // NW
    gamma2 = gamma.reshape(1, d)
    p2 = partial.reshape(m_total, d)

    def body(
        p_ref,
        g_ref,
        out_ref,
        zrecv,
        zsend_sems,
        zrecv_sems,
        ysend_sems,
        yrecv_sems,
        xsend_sems,
        xrecv_sems,
        gather_ready,
    ):
        my_x = lax.axis_index("x")
        my_y = lax.axis_index("y")
        my_z = lax.axis_index("z")
        other_x = 1 - my_x
        s_idx = my_x * NY + my_y

        barrier_sem = pltpu.get_barrier_semaphore()
        for r in range(1, NZ):
            pl.semaphore_signal(
                barrier_sem,
                inc=1,
                device_id=(my_x, my_y, (my_z + r) % NZ),
                device_id_type=pl.DeviceIdType.MESH,
            )
        for r in range(1, NY):
            pl.semaphore_signal(
                gather_ready,
                inc=1,
                device_id=(my_x, (my_y + r) % NY, my_z),
                device_id_type=pl.DeviceIdType.MESH,
            )
        pl.semaphore_signal(
            gather_ready,
            inc=1,
            device_id=(other_x, my_y, my_z),
            device_id_type=pl.DeviceIdType.MESH,
        )
        pl.semaphore_wait(barrier_sem, NZ - 1)

        zflows = [[] for _ in range(NW)]
        for w in range(NW):
            for r in range(1, NZ):
                c = (my_z + r) % NZ
                src_rows = pl.ds(
                    c * m_per + s_idx * m_strip + w * m_wave, m_wave
                )
                rdma = pltpu.make_async_remote_copy(
                    src_ref=p_ref.at[src_rows, :],
                    dst_ref=zrecv.at[w, r - 1],
                    send_sem=zsend_sems.at[w, r - 1],
                    recv_sem=zrecv_sems.at[w, r - 1],
                    device_id=(my_x, my_y, c),
                    device_id_type=pl.DeviceIdType.MESH,
                )
                rdma.start()
                zflows[w].append(rdma)

        yflows = [[] for _ in range(NW)]
        xflows = [[] for _ in range(NW)]
        for w in range(NW):
            wave_rows = pl.ds(s_idx * m_strip + w * m_wave, m_wave)
            acc = p_ref[
                pl.ds(my_z * m_per + s_idx * m_strip + w * m_wave, m_wave), :
            ]
            for r in range(1, NZ):
                zflows[w][r - 1].wait_recv()
                acc = acc + zrecv[w, r - 1]
            rms = jnp.sqrt(jnp.mean(acc * acc, axis=1, keepdims=True) + EPS)
            out_ref[wave_rows, :] = acc / rms * g_ref[:, :]

            if w == 0:
                pl.semaphore_wait(gather_ready, NY - 1 + 1)

            for r in range(1, NY):
                rdma = pltpu.make_async_remote_copy(
                    src_ref=out_ref.at[wave_rows, :],
                    dst_ref=out_ref.at[wave_rows, :],
                    send_sem=ysend_sems.at[w, r - 1],
                    recv_sem=yrecv_sems.at[w, r - 1],
                    device_id=(my_x, (my_y + r) % NY, my_z),
                    device_id_type=pl.DeviceIdType.MESH,
                )
                rdma.start()
                yflows[w].append(rdma)
            x0 = pltpu.make_async_remote_copy(
                src_ref=out_ref.at[wave_rows, :],
                dst_ref=out_ref.at[wave_rows, :],
                send_sem=xsend_sems.at[w, 0],
                recv_sem=xrecv_sems.at[w, 0],
                device_id=(other_x, my_y, my_z),
                device_id_type=pl.DeviceIdType.MESH,
            )
            x0.start()
            xflows[w].append(x0)

        for w in range(NW):
            for r in range(1, NY):
                yflows[w][r - 1].wait_recv()
                src_y = (my_y - r) % NY
                rows = pl.ds(
                    (my_x * NY + src_y) * m_strip + w * m_wave, m_wave
                )
                fwd = pltpu.make_async_remote_copy(
                    src_ref=out_ref.at[rows, :],
                    dst_ref=out_ref.at[rows, :],
                    send_sem=xsend_sems.at[w, r],
                    recv_sem=xrecv_sems.at[w, r],
                    device_id=(other_x, my_y, my_z),
                    device_id_type=pl.DeviceIdType.MESH,
                )
                fwd.start()
                xflows[w].append(fwd)

        for w in range(NW):
            for k in range(4):
                xflows[w][k].wait_recv()
        for w in range(NW):
            for r in range(1, NZ):
                zflows[w][r - 1].wait_send()
            for r in range(1, NY):
                yflows[w][r - 1].wait_send()
            for k in range(4):
                xflows[w][k].wait_send()

    return pl.pallas_call(
        body,
        out_shape=jax.ShapeDtypeStruct((m_per, d), jnp.float32),
        in_specs=[
            pl.BlockSpec(memory_space=pltpu.VMEM),
            pl.BlockSpec(memory_space=pltpu.VMEM),
        ],
        out_specs=pl.BlockSpec(memory_space=pltpu.VMEM),
        scratch_shapes=[
            pltpu.VMEM((NW, NZ - 1, m_wave, d), jnp.float32),
            pltpu.SemaphoreType.DMA((NW, NZ - 1)),
            pltpu.SemaphoreType.DMA((NW, NZ - 1)),
            pltpu.SemaphoreType.DMA((NW, NY - 1)),
            pltpu.SemaphoreType.DMA((NW, NY - 1)),
            pltpu.SemaphoreType.DMA((NW, 4)),
            pltpu.SemaphoreType.DMA((NW, 4)),
            pltpu.SemaphoreType.REGULAR,
        ],
        compiler_params=pltpu.CompilerParams(collective_id=0),
    )(p2, gamma2)


# device time: 23684 ns/iter; 1.9845x vs baseline; 1.0159x over previous
import jax
import jax.numpy as jnp
from jax import lax
from jax.experimental import pallas as pl
from jax.experimental.pallas import tpu as pltpu

NZ = 4
NY = 4
NW = 4
EPS = 1e-6


def kernel(partial, gamma):
    _, m_total, d = partial.shape
    m_per = m_total // NZ
    m_strip = m_per // (2 * NY)
    m_wave = m_strip // NW
    gamma2 = gamma.reshape(1, d)
    p2 = partial.reshape(m_total, d)

    def body(
        p_ref,
        g_ref,
        out_ref,
        zrecv,
        zsend_sems,
        zrecv_sems,
        ysend_sems,
        yrecv_sems,
        xsend_sems,
        xrecv_sems,
        gather_ready,
    ):
        my_x = lax.axis_index("x")
        my_y = lax.axis_index("y")
        my_z = lax.axis_index("z")
        other_x = 1 - my_x
        s_idx = my_x * NY + my_y

        barrier_sem = pltpu.get_barrier_semaphore()
        for r in range(1, NZ):
            pl.semaphore_signal(
                barrier_sem,
                inc=1,
                device_id=(my_x, my_y, (my_z + r) % NZ),
                device_id_type=pl.DeviceIdType.MESH,
            )
        for r in range(1, NY):
            pl.semaphore_signal(
                gather_ready,
                inc=1,
                device_id=(my_x, (my_y + r) % NY, my_z),
                device_id_type=pl.DeviceIdType.MESH,
            )
        pl.semaphore_signal(
            gather_ready,
            inc=1,
            device_id=(other_x, my_y, my_z),
            device_id_type=pl.DeviceIdType.MESH,
        )
        pl.semaphore_wait(barrier_sem, NZ - 1)

        zflows = [[] for _ in range(NW)]
        for w in range(NW):
            for r in range(1, NZ):
                c = (my_z + r) % NZ
                src_rows = pl.ds(
                    c * m_per + s_idx * m_strip + w * m_wave, m_wave
                )
                rdma = pltpu.make_async_remote_copy(
                    src_ref=p_ref.at[src_rows, :],
                    dst_ref=zrecv.at[w, r - 1],
                    send_sem=zsend_sems.at[w, r - 1],
                    recv_sem=zrecv_sems.at[w, r - 1],
                    device_id=(my_x, my_y, c),
                    device_id_type=pl.DeviceIdType.MESH,
                )
                rdma.start()
                zflows[w].append(rdma)

        yflows = [[] for _ in range(NW)]
        xflows = [[] for _ in range(NW)]
        for w in range(NW):
            wave_rows = pl.ds(s_idx * m_strip + w * m_wave, m_wave)
            acc = p_ref[
                pl.ds(my_z * m_per + s_idx * m_strip + w * m_wave, m_wave), :
            ]
            for r in range(1, NZ):
                zflows[w][r - 1].wait_recv()
                acc = acc + zrecv[w, r - 1]
            rms = jnp.sqrt(jnp.mean(acc * acc, axis=1, keepdims=True) + EPS)
            out_ref[wave_rows, :] = acc / rms * g_ref[:, :]

            if w == 0:
                pl.semaphore_wait(gather_ready, NY - 1 + 1)

            x0 = pltpu.make_async_remote_copy(
                src_ref=out_ref.at[wave_rows, :],
                dst_ref=out_ref.at[wave_rows, :],
                send_sem=xsend_sems.at[w, 0],
                recv_sem=xrecv_sems.at[w, 0],
                device_id=(other_x, my_y, my_z),
                device_id_type=pl.DeviceIdType.MESH,
            )
            x0.start()
            xflows[w].append(x0)
            for r in range(1, NY):
                rdma = pltpu.make_async_remote_copy(
                    src_ref=out_ref.at[wave_rows, :],
                    dst_ref=out_ref.at[wave_rows, :],
                    send_sem=ysend_sems.at[w, r - 1],
                    recv_sem=yrecv_sems.at[w, r - 1],
                    device_id=(my_x, (my_y + r) % NY, my_z),
                    device_id_type=pl.DeviceIdType.MESH,
                )
                rdma.start()
                yflows[w].append(rdma)
            if w == NW - 1:
                for r in range(1, NY):
                    diag = pltpu.make_async_remote_copy(
                        src_ref=out_ref.at[wave_rows, :],
                        dst_ref=out_ref.at[wave_rows, :],
                        send_sem=xsend_sems.at[w, r],
                        recv_sem=xrecv_sems.at[w, r],
                        device_id=(other_x, (my_y + r) % NY, my_z),
                        device_id_type=pl.DeviceIdType.MESH,
                    )
                    diag.start()
                    xflows[w].append(diag)

        for w in range(NW - 1):
            for r in range(1, NY):
                yflows[w][r - 1].wait_recv()
                src_y = (my_y - r) % NY
                rows = pl.ds(
                    (my_x * NY + src_y) * m_strip + w * m_wave, m_wave
                )
                fwd = pltpu.make_async_remote_copy(
                    src_ref=out_ref.at[rows, :],
                    dst_ref=out_ref.at[rows, :],
                    send_sem=xsend_sems.at[w, r],
                    recv_sem=xrecv_sems.at[w, r],
                    device_id=(other_x, my_y, my_z),
                    device_id_type=pl.DeviceIdType.MESH,
                )
                fwd.start()
                xflows[w].append(fwd)
        for r in range(1, NY):
            yflows[NW - 1][r - 1].wait_recv()

        for w in range(NW):
            for k in range(4):
                xflows[w][k].wait_recv()
        for w in range(NW):
            for r in range(1, NZ):
                zflows[w][r - 1].wait_send()
            for r in range(1, NY):
                yflows[w][r - 1].wait_send()
            for k in range(4):
                xflows[w][k].wait_send()

    return pl.pallas_call(
        body,
        out_shape=jax.ShapeDtypeStruct((m_per, d), jnp.float32),
        in_specs=[
            pl.BlockSpec(memory_space=pltpu.VMEM),
            pl.BlockSpec(memory_space=pltpu.VMEM),
        ],
        out_specs=pl.BlockSpec(memory_space=pltpu.VMEM),
        scratch_shapes=[
            pltpu.VMEM((NW, NZ - 1, m_wave, d), jnp.float32),
            pltpu.SemaphoreType.DMA((NW, NZ - 1)),
            pltpu.SemaphoreType.DMA((NW, NZ - 1)),
            pltpu.SemaphoreType.DMA((NW, NY - 1)),
            pltpu.SemaphoreType.DMA((NW, NY - 1)),
            pltpu.SemaphoreType.DMA((NW, 4)),
            pltpu.SemaphoreType.DMA((NW, 4)),
            pltpu.SemaphoreType.REGULAR,
        ],
        compiler_params=pltpu.CompilerParams(collective_id=0),
    )(p2, gamma2)
